# baseline (speedup 1.0000x reference)
import functools

import jax
import jax.numpy as jnp
from jax.experimental import pallas as pl
from jax.experimental.pallas import tpu as pltpu

_KP1 = 4


def _fourth_smallest_sq(d2):
    t = d2
    m = None
    for it in range(_KP1):
        m = jnp.min(t, axis=1, keepdims=True)
        if it < _KP1 - 1:
            t = jnp.where(t <= m, jnp.inf, t)
    return m


def _body(real_ref, fake_ref, rr_ref, rf_ref, met_ref,
          nr_ref, nf_ref, r2r_ref, r2f_ref, prec_ref, rec_ref,
          ga_ref, gb_ref,
          *, bm, nb, n):
    p = pl.program_id(0)
    i = pl.program_id(1)
    parity = jax.lax.rem(i, 2)
    im = jnp.minimum(i, nb - 1)
    j = jnp.maximum(i - 1, 0)
    slm = pl.ds(im * bm, bm)
    slj = pl.ds(j * bm, bm)

    @pl.when(p == 0)
    def _norms():
        rrow = real_ref[slm, :].astype(jnp.float32)
        nr_ref[0, slm] = jnp.sum(rrow * rrow, axis=1)
        frow = fake_ref[slm, :].astype(jnp.float32)
        nf_ref[0, slm] = jnp.sum(frow * frow, axis=1)

    def _gram(rows_ref, cols_ref, gw_ref):
        gw_ref[...] = jax.lax.dot_general(
            rows_ref[slm, :], cols_ref[...],
            dimension_numbers=(((1,), (1,)), ((), ())),
            preferred_element_type=jnp.float32)

    def _d2_prev(gr_ref, rownorm_ref, colnorm_ref):
        xn = rownorm_ref[0, slj].reshape(bm, 1)
        return xn + colnorm_ref[...] - 2.0 * gr_ref[...]

    def _radii_tail(gr_ref, norm_ref, radii_out_ref, r2_out_ref):
        d2 = _d2_prev(gr_ref, norm_ref, norm_ref)
        v4 = _fourth_smallest_sq(d2)
        r2 = jnp.maximum(v4, 1e-12)
        r2_out_ref[0, slj] = r2[:, 0]
        radii_out_ref[0, slj] = jnp.sqrt(r2)[:, 0]

    def _cross_tail(gr_ref):
        d2 = _d2_prev(gr_ref, nf_ref, nr_ref)
        c2 = jnp.maximum(d2, 1e-12)
        within_real = (c2 <= r2r_ref[...]).astype(jnp.float32)
        prec_ref[0, slj] = jnp.max(within_real, axis=1)
        r2f_block = r2f_ref[0, slj].reshape(bm, 1)
        within_fake = (c2 <= r2f_block).astype(jnp.float32)
        rec_part = jnp.max(within_fake, axis=0, keepdims=True)
        rec_ref[...] = jnp.where(
            i <= 1, rec_part, jnp.maximum(rec_ref[...], rec_part))

    for parity_val in (0, 1):
        gw, gr = (ga_ref, gb_ref) if parity_val == 0 else (gb_ref, ga_ref)

        @pl.when((p == 1) & (parity == parity_val))
        def _(gw=gw, gr=gr):
            _gram(real_ref, real_ref, gw)
            _radii_tail(gr, nr_ref, rr_ref, r2r_ref)

        @pl.when((p == 2) & (parity == parity_val))
        def _(gw=gw, gr=gr):
            _gram(fake_ref, fake_ref, gw)
            _radii_tail(gr, nf_ref, rf_ref, r2f_ref)

        @pl.when((p == 3) & (parity == parity_val))
        def _(gw=gw, gr=gr):
            _gram(fake_ref, real_ref, gw)
            _cross_tail(gr)

    @pl.when((p == 3) & (i == nb))
    def _metrics():
        precision = jnp.sum(prec_ref[...]) / n
        recall = jnp.sum(rec_ref[...]) / n
        lane = jax.lax.broadcasted_iota(jnp.int32, (1, 128), 1)
        met_ref[...] = jnp.where(
            lane == 0, precision, jnp.where(lane == 1, recall, 0.0))


def kernel(real_feats, fake_feats):
    n, d = real_feats.shape
    bm = 256 if n % 256 == 0 else n
    nb = n // bm

    real_bf = real_feats.astype(jnp.bfloat16)
    fake_bf = fake_feats.astype(jnp.bfloat16)

    body = functools.partial(_body, bm=bm, nb=nb, n=n)

    full = pl.BlockSpec((n, d), lambda p, i: (0, 0))
    vec = pl.BlockSpec((1, n), lambda p, i: (0, 0))
    met = pl.BlockSpec((1, 128), lambda p, i: (0, 0))

    rr, rf, metrics = pl.pallas_call(
        body,
        grid=(4, nb + 1),
        in_specs=[full, full],
        out_specs=[vec, vec, met],
        out_shape=[
            jax.ShapeDtypeStruct((1, n), jnp.float32),
            jax.ShapeDtypeStruct((1, n), jnp.float32),
            jax.ShapeDtypeStruct((1, 128), jnp.float32),
        ],
        scratch_shapes=[
            pltpu.VMEM((1, n), jnp.float32),
            pltpu.VMEM((1, n), jnp.float32),
            pltpu.VMEM((1, n), jnp.float32),
            pltpu.VMEM((1, n), jnp.float32),
            pltpu.VMEM((1, n), jnp.float32),
            pltpu.VMEM((1, n), jnp.float32),
            pltpu.VMEM((bm, n), jnp.float32),
            pltpu.VMEM((bm, n), jnp.float32),
        ],
        compiler_params=pltpu.CompilerParams(
            dimension_semantics=("arbitrary", "arbitrary")),
    )(real_bf, fake_bf)

    return jnp.concatenate(
        [metrics[0, :2], rr[0, :], rf[0, :]])


# speedup vs baseline: 1.1571x; 1.1571x over previous
import functools

import jax
import jax.numpy as jnp
from jax.experimental import pallas as pl
from jax.experimental.pallas import tpu as pltpu

_C = 128


def _fourth_smallest_streamed(g, yn, xn, bm, n):
    nc = n // _C
    inf = jnp.full((bm, _C), jnp.inf, dtype=jnp.float32)
    m1, m2, m3, m4 = inf, inf, inf, inf
    for c in range(nc):
        v = yn[:, c * _C:(c + 1) * _C] - 2.0 * g[:, c * _C:(c + 1) * _C]
        hi = jnp.maximum(m1, v)
        m1 = jnp.minimum(m1, v)
        hi2 = jnp.maximum(m2, hi)
        m2 = jnp.minimum(m2, hi)
        hi3 = jnp.maximum(m3, hi2)
        m3 = jnp.minimum(m3, hi2)
        m4 = jnp.minimum(m4, hi3)
    cand = jnp.concatenate([m1, m2, m3, m4], axis=1)
    m = None
    for it in range(4):
        m = jnp.min(cand, axis=1, keepdims=True)
        if it < 3:
            cand = jnp.where(cand <= m, jnp.inf, cand)
    return m + xn


def _body(real_ref, fake_ref, rr_ref, rf_ref, met_ref,
          nr_ref, nf_ref, r2r_ref, r2f_ref, prec_ref, rec_ref,
          *, bm, nb, n):
    p = pl.program_id(0)
    i = pl.program_id(1)
    sl = pl.ds(i * bm, bm)

    @pl.when(p == 0)
    def _norms():
        rrow = real_ref[sl, :].astype(jnp.float32)
        nr_ref[0, sl] = jnp.sum(rrow * rrow, axis=1)
        frow = fake_ref[sl, :].astype(jnp.float32)
        nf_ref[0, sl] = jnp.sum(frow * frow, axis=1)

    def _gram(rows_ref, cols_ref):
        return jax.lax.dot_general(
            rows_ref[sl, :], cols_ref[...],
            dimension_numbers=(((1,), (1,)), ((), ())),
            preferred_element_type=jnp.float32)

    def _radii_phase(src_ref, norm_ref, radii_out_ref, r2_out_ref):
        g = _gram(src_ref, src_ref)
        xn = norm_ref[0, sl].reshape(bm, 1)
        v4 = _fourth_smallest_streamed(g, norm_ref[...], xn, bm, n)
        r2 = jnp.maximum(v4, 1e-12)
        r2_out_ref[0, sl] = r2[:, 0]
        radii_out_ref[0, sl] = jnp.sqrt(r2)[:, 0]

    @pl.when(p == 1)
    def _real_radii():
        _radii_phase(real_ref, nr_ref, rr_ref, r2r_ref)

    @pl.when(p == 2)
    def _fake_radii():
        _radii_phase(fake_ref, nf_ref, rf_ref, r2f_ref)

    @pl.when(p == 3)
    def _cross():
        g = _gram(fake_ref, real_ref)
        xn = nf_ref[0, sl].reshape(bm, 1)
        r2f_block = r2f_ref[0, sl].reshape(bm, 1)
        prec_acc = jnp.zeros((bm, _C), dtype=jnp.float32)
        rec_chunks = []
        for c in range(n // _C):
            d2 = (xn + nr_ref[:, c * _C:(c + 1) * _C]
                  - 2.0 * g[:, c * _C:(c + 1) * _C])
            within_real = (d2 <= r2r_ref[:, c * _C:(c + 1) * _C])
            prec_acc = jnp.maximum(prec_acc, within_real.astype(jnp.float32))
            within_fake = (d2 <= r2f_block).astype(jnp.float32)
            rec_chunks.append(jnp.max(within_fake, axis=0, keepdims=True))
        prec_ref[0, sl] = jnp.max(prec_acc, axis=1)
        rec_part = jnp.concatenate(rec_chunks, axis=1)
        rec_ref[...] = jnp.where(
            i == 0, rec_part, jnp.maximum(rec_ref[...], rec_part))

        @pl.when(i == nb - 1)
        def _():
            precision = jnp.sum(prec_ref[...]) / n
            recall = jnp.sum(rec_ref[...]) / n
            lane = jax.lax.broadcasted_iota(jnp.int32, (1, 128), 1)
            met_ref[...] = jnp.where(
                lane == 0, precision, jnp.where(lane == 1, recall, 0.0))


def kernel(real_feats, fake_feats):
    n, d = real_feats.shape
    bm = 256 if n % 256 == 0 else n
    nb = n // bm

    real_bf = real_feats.astype(jnp.bfloat16)
    fake_bf = fake_feats.astype(jnp.bfloat16)

    body = functools.partial(_body, bm=bm, nb=nb, n=n)

    full = pl.BlockSpec((n, d), lambda p, i: (0, 0))
    vec = pl.BlockSpec((1, n), lambda p, i: (0, 0))
    met = pl.BlockSpec((1, 128), lambda p, i: (0, 0))

    rr, rf, metrics = pl.pallas_call(
        body,
        grid=(4, nb),
        in_specs=[full, full],
        out_specs=[vec, vec, met],
        out_shape=[
            jax.ShapeDtypeStruct((1, n), jnp.float32),
            jax.ShapeDtypeStruct((1, n), jnp.float32),
            jax.ShapeDtypeStruct((1, 128), jnp.float32),
        ],
        scratch_shapes=[
            pltpu.VMEM((1, n), jnp.float32),
            pltpu.VMEM((1, n), jnp.float32),
            pltpu.VMEM((1, n), jnp.float32),
            pltpu.VMEM((1, n), jnp.float32),
            pltpu.VMEM((1, n), jnp.float32),
            pltpu.VMEM((1, n), jnp.float32),
        ],
        compiler_params=pltpu.CompilerParams(
            dimension_semantics=("arbitrary", "arbitrary")),
    )(real_bf, fake_bf)

    return jnp.concatenate(
        [metrics[0, :2], rr[0, :], rf[0, :]])
